# baseline (speedup 1.0000x reference)
import jax
import jax.numpy as jnp
from jax import lax
from jax.experimental import pallas as pl
from jax.experimental.pallas import tpu as pltpu
from jax.experimental.pallas import tpu_sc as plsc

_NU = 50000
_NI = 50000
_EMB = 32
_N_LAYERS = 3
_N_INTER = 1600000
_BATCH = 4096

_NC = 2
_NS = 16
_LANES = 128
_K = 2
_NSL = 3
_PR = 51200
_STRIPE = _PR // _NS
_G = 2 * _PR
_DUMMY = _NU

_EDGES_PER_BURST = _K * _LANES
_NB = -(-_N_INTER // (_NS * _EDGES_PER_BURST))
_EP = _NB * _NS * _EDGES_PER_BURST
_ROWS = _EP // _LANES
_RT = _ROWS // _NS

_MESH = plsc.VectorSubcoreMesh(core_axis_name="c", subcore_axis_name="s")
_SC_PARAMS = pltpu.CompilerParams(
    use_tc_tiling_on_sc=False, internal_scratch_in_bytes=0)


def _zero_stripe(acc, zbuf, s):
    def _zb(r, carry):
        zbuf[r, pl.ds(0, 16)] = jnp.zeros((16,), jnp.float32)
        zbuf[r, pl.ds(16, 16)] = jnp.zeros((16,), jnp.float32)
        return carry

    lax.fori_loop(0, _LANES, _zb, 0)

    def _zc(r, carry):
        pltpu.sync_copy(zbuf, acc.at[pl.ds(s * _STRIPE + r * _LANES, _LANES)])
        return carry

    lax.fori_loop(0, _STRIPE // _LANES, _zc, 0)


def _spmm_body(table, uidx, iidx, out, acc, sidx, didx, rows, gsem, ssem, isem):
    c = lax.axis_index("c")
    s = lax.axis_index("s")
    _zero_stripe(acc, rows.at[0, 0], s)
    plsc.subcore_barrier()

    row_base = s * _RT

    def _run(src_hbm, dst_hbm, tci):
        tbl = table.at[tci]

        def _fire_gathers(slot):
            for j in range(_K):
                pltpu.async_copy(tbl.at[sidx.at[slot, j]], rows.at[slot, j],
                                 gsem)

        def _wait_gathers(slot):
            for j in range(_K):
                pltpu.make_async_copy(
                    tbl.at[sidx.at[slot, j]], rows.at[slot, j], gsem).wait()

        def _fire_scatters(slot):
            for j in range(_K):
                pltpu.async_copy(rows.at[slot, j], acc.at[didx.at[slot, j]],
                                 ssem, add=True)

        def _wait_scatters(slot):
            for j in range(_K):
                pltpu.make_async_copy(
                    rows.at[slot, j], acc.at[didx.at[slot, j]], ssem).wait()

        def _load_idx(b, slot):
            i1 = pltpu.async_copy(
                src_hbm.at[pl.ds(row_base + b * _K, _K)], sidx.at[slot], isem)
            i2 = pltpu.async_copy(
                dst_hbm.at[pl.ds(row_base + b * _K, _K)], didx.at[slot], isem)
            i1.wait()
            i2.wait()

        for p in range(2):
            _load_idx(p, p)
            _fire_gathers(p)

        def _burst(b, carry):
            sb = lax.rem(b, _NSL)
            sp = lax.rem(b + 2, _NSL)

            @pl.when(b >= 1)
            def _():
                _wait_scatters(sp)

            _load_idx(b + 2, sp)
            _fire_gathers(sp)
            _wait_gathers(sb)
            _fire_scatters(sb)
            return carry

        lax.fori_loop(0, _NB - 2, _burst, 0)

        for t in (_NB - 2, _NB - 1):
            _wait_gathers(t % _NSL)
            _fire_scatters(t % _NSL)
        for t in (_NB - 3, _NB - 2, _NB - 1):
            _wait_scatters(t % _NSL)

    @pl.when(c == 0)
    def _():
        _run(iidx, uidx, 1)

    @pl.when(c == 1)
    def _():
        _run(uidx, iidx, 0)

    plsc.subcore_barrier()
    pltpu.sync_copy(acc.at[pl.ds(s * _STRIPE, _STRIPE)],
                    out.at[c, pl.ds(s * _STRIPE, _STRIPE)])


_spmm = pl.kernel(
    _spmm_body,
    out_type=jax.ShapeDtypeStruct((_NC, _PR, _EMB), jnp.float32),
    mesh=_MESH,
    compiler_params=_SC_PARAMS,
    scratch_types=[
        pltpu.VMEM_SHARED((_PR, _EMB), jnp.float32),
        pltpu.VMEM((_NSL, _K, _LANES), jnp.int32),
        pltpu.VMEM((_NSL, _K, _LANES), jnp.int32),
        pltpu.VMEM((_NSL, _K, _LANES, _EMB), jnp.float32),
        pltpu.SemaphoreType.DMA,
        pltpu.SemaphoreType.DMA,
        pltpu.SemaphoreType.DMA,
    ],
    name="lightgcn_spmm_sc",
)


_K2 = 32
_NB2 = -(-_N_INTER // (_NS * _K2 * _LANES))
_EP2 = _NB2 * _NS * _K2 * _LANES
_ROWS2 = _EP2 // _LANES
_RT2 = _ROWS2 // _NS


_DW = 16


def _deg_body(uidx, iidx, out, acc, didx, obuf, zbuf, wbuf, dbuf, ssem, isem):
    c = lax.axis_index("c")
    s = lax.axis_index("s")

    def _zb(r, carry):
        zbuf[r, pl.ds(0, 16)] = jnp.zeros((16,), jnp.float32)
        return carry

    lax.fori_loop(0, _LANES, _zb, 0)

    def _zc(r, carry):
        pltpu.sync_copy(zbuf, acc.at[pl.ds(s * _STRIPE + r * _LANES, _LANES)])
        return carry

    lax.fori_loop(0, _STRIPE // _LANES, _zc, 0)

    def _ones(r, carry):
        obuf[r, pl.ds(0, 16)] = jnp.ones((16,), jnp.float32)
        return carry

    lax.fori_loop(0, _LANES, _ones, 0)
    plsc.subcore_barrier()

    row_base = s * _RT2

    def _run(dst_hbm):
        def _fire(slot):
            for j in range(_K2):
                pltpu.async_copy(obuf, acc.at[didx.at[slot, j]], ssem,
                                 add=True)

        def _wait(slot):
            for j in range(_K2):
                pltpu.make_async_copy(obuf, acc.at[didx.at[slot, j]],
                                      ssem).wait()

        pltpu.sync_copy(dst_hbm.at[pl.ds(row_base, _K2)], didx.at[0])

        def _burst(b, carry):
            buf = lax.rem(b, 2)
            nxt = 1 - buf

            @pl.when(b >= 1)
            def _():
                _wait(nxt)

            il = pltpu.async_copy(
                dst_hbm.at[pl.ds(row_base + (b + 1) * _K2, _K2)],
                didx.at[nxt], isem)
            _fire(buf)
            il.wait()
            return carry

        lax.fori_loop(0, _NB2 - 1, _burst, 0)
        last = (_NB2 - 1) % 2
        _fire(last)
        _wait(last)
        _wait(1 - last)

    @pl.when(c == 0)
    def _():
        _run(uidx)

    @pl.when(c == 1)
    def _():
        _run(iidx)

    plsc.subcore_barrier()

    _DCH = 640

    def _dup_chunk(t, carry):
        base = s * _STRIPE + t * _DCH
        pltpu.sync_copy(acc.at[pl.ds(base, _DCH)], wbuf)

        def _dup(r, carry2):
            v = wbuf[r, pl.ds(0, 16)]
            dbuf[r, pl.ds(0, 16)] = v
            dbuf[r, pl.ds(16, 16)] = v
            return carry2

        lax.fori_loop(0, _DCH, _dup, 0)
        pltpu.sync_copy(dbuf, out.at[c, pl.ds(base, _DCH)])
        return carry

    lax.fori_loop(0, _STRIPE // _DCH, _dup_chunk, 0)


_deg = pl.kernel(
    _deg_body,
    out_type=jax.ShapeDtypeStruct((_NC, _PR, _EMB), jnp.float32),
    mesh=_MESH,
    compiler_params=_SC_PARAMS,
    scratch_types=[
        pltpu.VMEM_SHARED((_PR, _DW), jnp.float32),
        pltpu.VMEM((2, _K2, _LANES), jnp.int32),
        pltpu.VMEM((_LANES, _DW), jnp.float32),
        pltpu.VMEM((_LANES, _DW), jnp.float32),
        pltpu.VMEM((640, _DW), jnp.float32),
        pltpu.VMEM((640, _EMB), jnp.float32),
        pltpu.SemaphoreType.DMA,
        pltpu.SemaphoreType.DMA,
    ],
    name="lightgcn_deg_sc",
)


def _gather_rows_body(tbl_e, tbl_s, tbl_d, idx, out, ibuf, ebuf, sbuf, dbuf,
                      gsem):
    c = lax.axis_index("c")
    s = lax.axis_index("s")
    wid = s * _NC + c
    nrows = 3
    pltpu.sync_copy(idx.at[pl.ds(wid * nrows, nrows)], ibuf)
    gathers = []
    for j in range(nrows):
        gathers.append(pltpu.async_copy(tbl_e.at[ibuf.at[j]], ebuf.at[j], gsem))
        gathers.append(pltpu.async_copy(tbl_s.at[ibuf.at[j]], sbuf.at[j], gsem))
        gathers.append(pltpu.async_copy(tbl_d.at[ibuf.at[j]], dbuf.at[j], gsem))
    for g in gathers:
        g.wait()

    def _combine(r, carry):
        for j in range(nrows):
            for h in (0, 16):
                e = ebuf[j, r, pl.ds(h, 16)]
                sv = sbuf[j, r, pl.ds(h, 16)]
                d = dbuf[j, r, pl.ds(h, 16)]
                ebuf[j, r, pl.ds(h, 16)] = (e + d * sv) * 0.25
        return carry

    lax.fori_loop(0, _LANES, _combine, 0)
    for j in range(nrows):
        pltpu.sync_copy(ebuf.at[j],
                        out.at[pl.ds(wid * nrows * _LANES + j * _LANES, _LANES)])


_gather_rows = pl.kernel(
    _gather_rows_body,
    out_type=jax.ShapeDtypeStruct((3 * _BATCH, _EMB), jnp.float32),
    mesh=_MESH,
    compiler_params=_SC_PARAMS,
    scratch_types=[
        pltpu.VMEM((3, _LANES), jnp.int32),
        pltpu.VMEM((3, _LANES, _EMB), jnp.float32),
        pltpu.VMEM((3, _LANES, _EMB), jnp.float32),
        pltpu.VMEM((3, _LANES, _EMB), jnp.float32),
        pltpu.SemaphoreType.DMA,
    ],
    name="lightgcn_batch_gather_sc",
)


_EW_ROWS = _G * _EMB // 128
_EW_BLOCK = _EW_ROWS // 8


def _ew_call(body, n_in, n_out):
    return pl.pallas_call(
        body,
        grid=(8,),
        in_specs=[pl.BlockSpec((_EW_BLOCK, 128), lambda i: (i, 0))] * n_in,
        out_specs=(pl.BlockSpec((_EW_BLOCK, 128), lambda i: (i, 0)),) * n_out,
        out_shape=(jax.ShapeDtypeStruct((_EW_ROWS, 128), jnp.float32),) * n_out,
    )


def _d_scale_body(deg_ref, e_ref, d_ref, eh_ref):
    d = lax.rsqrt(deg_ref[...] + 1e-9)
    d = jnp.where(jnp.isinf(d), 0.0, d)
    d_ref[...] = d
    eh_ref[...] = d * e_ref[...]


def _boundary_body(e_ref, s_ref, d_ref, en_ref, eh_ref):
    en = e_ref[...] + d_ref[...] * s_ref[...]
    en_ref[...] = en
    eh_ref[...] = d_ref[...] * en


def kernel(E0, users, pos_items, neg_items, user_idx, item_idx):
    f32 = jnp.float32
    i32 = jnp.int32
    ui = user_idx.astype(i32)
    ii = item_idx.astype(i32)

    pad_idx = jnp.full((_EP - _N_INTER,), _DUMMY, i32)
    uidx = jnp.concatenate([ui, pad_idx]).reshape(_ROWS, _LANES)
    iidx = jnp.concatenate([ii, pad_idx]).reshape(_ROWS, _LANES)
    pad2_idx = jnp.full((_EP2 - _N_INTER,), _DUMMY, i32)
    uidx2 = jnp.concatenate([ui, pad2_idx]).reshape(_ROWS2, _LANES)
    iidx2 = jnp.concatenate([ii, pad2_idx]).reshape(_ROWS2, _LANES)

    zpad = jnp.zeros((_PR - _NU, _EMB), f32)
    E_2 = jnp.stack([
        jnp.concatenate([E0[:_NU].astype(f32), zpad]),
        jnp.concatenate([E0[_NU:].astype(f32), zpad]),
    ])

    ew = lambda x: x.reshape(_EW_ROWS, 128)
    un2 = lambda x: x.reshape(_NC, _PR, _EMB)

    deg_g = ew(_deg(uidx2, iidx2))
    e_cur = ew(E_2)
    d_g, ehat = _ew_call(_d_scale_body, 2, 2)(deg_g, e_cur)
    for layer in range(_N_LAYERS):
        S = ew(_spmm(un2(ehat), uidx, iidx))
        if layer < _N_LAYERS - 1:
            e_cur, ehat = _ew_call(_boundary_body, 3, 2)(e_cur, S, d_g)

    idx_all = jnp.concatenate([
        users.astype(i32),
        pos_items.astype(i32) + _PR,
        neg_items.astype(i32) + _PR,
    ]).reshape(3 * _BATCH // _LANES, _LANES)
    res = _gather_rows(e_cur.reshape(_G, _EMB), S.reshape(_G, _EMB),
                       d_g.reshape(_G, _EMB), idx_all)
    return res[:_BATCH], res[_BATCH:2 * _BATCH], res[2 * _BATCH:]


# speedup vs baseline: 1.0514x; 1.0514x over previous
import jax
import jax.numpy as jnp
from jax import lax
from jax.experimental import pallas as pl
from jax.experimental.pallas import tpu as pltpu
from jax.experimental.pallas import tpu_sc as plsc

_NU = 50000
_NI = 50000
_EMB = 32
_N_LAYERS = 3
_N_INTER = 1600000
_BATCH = 4096

_NC = 2
_NS = 16
_LANES = 128
_K = 3
_PR = 51200
_STRIPE = _PR // _NS
_G = 2 * _PR
_DUMMY = _NU

_EDGES_PER_BURST = _K * _LANES
_NB = -(-_N_INTER // (_NS * _EDGES_PER_BURST))
_EP = _NB * _NS * _EDGES_PER_BURST
_ROWS = _EP // _LANES
_RT = _ROWS // _NS

_MESH = plsc.VectorSubcoreMesh(core_axis_name="c", subcore_axis_name="s")
_SC_PARAMS = pltpu.CompilerParams(
    use_tc_tiling_on_sc=False, internal_scratch_in_bytes=0)


def _zero_stripe(acc, zbuf, s):
    def _zb(r, carry):
        zbuf[r, pl.ds(0, 16)] = jnp.zeros((16,), jnp.float32)
        zbuf[r, pl.ds(16, 16)] = jnp.zeros((16,), jnp.float32)
        return carry

    lax.fori_loop(0, _LANES, _zb, 0)

    def _zc(r, carry):
        pltpu.sync_copy(zbuf, acc.at[pl.ds(s * _STRIPE + r * _LANES, _LANES)])
        return carry

    lax.fori_loop(0, _STRIPE // _LANES, _zc, 0)


def _spmm_body(table, uidx, iidx, out, acc, sidx, didx, rows, gsem, ssem, isem):
    c = lax.axis_index("c")
    s = lax.axis_index("s")
    _zero_stripe(acc, rows.at[0, 0], s)
    plsc.subcore_barrier()

    row_base = s * _RT

    def _run(src_hbm, dst_hbm, tci):
        tbl = table.at[tci]

        def _fire_gathers(slot):
            for j in range(_K):
                pltpu.async_copy(tbl.at[sidx.at[slot, j]], rows.at[slot, j],
                                 gsem)

        def _wait_gathers(slot):
            for j in range(_K):
                pltpu.make_async_copy(
                    tbl.at[sidx.at[slot, j]], rows.at[slot, j], gsem).wait()

        def _fire_scatters(slot):
            for j in range(_K):
                pltpu.async_copy(rows.at[slot, j], acc.at[didx.at[slot, j]],
                                 ssem, add=True)

        def _wait_scatters(slot):
            for j in range(_K):
                pltpu.make_async_copy(
                    rows.at[slot, j], acc.at[didx.at[slot, j]], ssem).wait()

        pltpu.sync_copy(src_hbm.at[pl.ds(row_base, _K)], sidx.at[0])
        pltpu.sync_copy(dst_hbm.at[pl.ds(row_base, _K)], didx.at[0])
        _fire_gathers(0)

        def _burst(b, carry):
            buf = lax.rem(b, 2)
            nxt = 1 - buf
            row1 = row_base + (b + 1) * _K

            @pl.when(b >= 1)
            def _():
                _wait_scatters(nxt)

            i1 = pltpu.async_copy(src_hbm.at[pl.ds(row1, _K)], sidx.at[nxt],
                                  isem)
            i2 = pltpu.async_copy(dst_hbm.at[pl.ds(row1, _K)], didx.at[nxt],
                                  isem)
            _wait_gathers(buf)
            _fire_scatters(buf)
            i1.wait()
            i2.wait()
            _fire_gathers(nxt)
            return carry

        lax.fori_loop(0, _NB - 1, _burst, 0)

        last = (_NB - 1) % 2
        _wait_gathers(last)
        _fire_scatters(last)
        _wait_scatters(last)
        _wait_scatters(1 - last)

    @pl.when(c == 0)
    def _():
        _run(iidx, uidx, 1)

    @pl.when(c == 1)
    def _():
        _run(uidx, iidx, 0)

    plsc.subcore_barrier()
    pltpu.sync_copy(acc.at[pl.ds(s * _STRIPE, _STRIPE)],
                    out.at[c, pl.ds(s * _STRIPE, _STRIPE)])


_spmm = pl.kernel(
    _spmm_body,
    out_type=jax.ShapeDtypeStruct((_NC, _PR, _EMB), jnp.float32),
    mesh=_MESH,
    compiler_params=_SC_PARAMS,
    scratch_types=[
        pltpu.VMEM_SHARED((_PR, _EMB), jnp.float32),
        pltpu.VMEM((2, _K, _LANES), jnp.int32),
        pltpu.VMEM((2, _K, _LANES), jnp.int32),
        pltpu.VMEM((2, _K, _LANES, _EMB), jnp.float32),
        pltpu.SemaphoreType.DMA,
        pltpu.SemaphoreType.DMA,
        pltpu.SemaphoreType.DMA,
    ],
    name="lightgcn_spmm_sc",
)


_K2 = 32
_NB2 = -(-_N_INTER // (_NS * _K2 * _LANES))
_EP2 = _NB2 * _NS * _K2 * _LANES
_ROWS2 = _EP2 // _LANES
_RT2 = _ROWS2 // _NS


_DW = 16


def _deg_body(uidx, iidx, out, acc, didx, obuf, zbuf, wbuf, dbuf, ssem, isem):
    c = lax.axis_index("c")
    s = lax.axis_index("s")

    def _zb(r, carry):
        zbuf[r, pl.ds(0, 16)] = jnp.zeros((16,), jnp.float32)
        return carry

    lax.fori_loop(0, _LANES, _zb, 0)

    def _zc(r, carry):
        pltpu.sync_copy(zbuf, acc.at[pl.ds(s * _STRIPE + r * _LANES, _LANES)])
        return carry

    lax.fori_loop(0, _STRIPE // _LANES, _zc, 0)

    def _ones(r, carry):
        obuf[r, pl.ds(0, 16)] = jnp.ones((16,), jnp.float32)
        return carry

    lax.fori_loop(0, _LANES, _ones, 0)
    plsc.subcore_barrier()

    row_base = s * _RT2

    def _run(dst_hbm):
        def _fire(slot):
            for j in range(_K2):
                pltpu.async_copy(obuf, acc.at[didx.at[slot, j]], ssem,
                                 add=True)

        def _wait(slot):
            for j in range(_K2):
                pltpu.make_async_copy(obuf, acc.at[didx.at[slot, j]],
                                      ssem).wait()

        pltpu.sync_copy(dst_hbm.at[pl.ds(row_base, _K2)], didx.at[0])

        def _burst(b, carry):
            buf = lax.rem(b, 2)
            nxt = 1 - buf

            @pl.when(b >= 1)
            def _():
                _wait(nxt)

            il = pltpu.async_copy(
                dst_hbm.at[pl.ds(row_base + (b + 1) * _K2, _K2)],
                didx.at[nxt], isem)
            _fire(buf)
            il.wait()
            return carry

        lax.fori_loop(0, _NB2 - 1, _burst, 0)
        last = (_NB2 - 1) % 2
        _fire(last)
        _wait(last)
        _wait(1 - last)

    @pl.when(c == 0)
    def _():
        _run(uidx)

    @pl.when(c == 1)
    def _():
        _run(iidx)

    plsc.subcore_barrier()

    _DCH = 640

    def _dup_chunk(t, carry):
        base = s * _STRIPE + t * _DCH
        pltpu.sync_copy(acc.at[pl.ds(base, _DCH)], wbuf)

        def _dup(r, carry2):
            v = wbuf[r, pl.ds(0, 16)]
            dbuf[r, pl.ds(0, 16)] = v
            dbuf[r, pl.ds(16, 16)] = v
            return carry2

        lax.fori_loop(0, _DCH, _dup, 0)
        pltpu.sync_copy(dbuf, out.at[c, pl.ds(base, _DCH)])
        return carry

    lax.fori_loop(0, _STRIPE // _DCH, _dup_chunk, 0)


_deg = pl.kernel(
    _deg_body,
    out_type=jax.ShapeDtypeStruct((_NC, _PR, _EMB), jnp.float32),
    mesh=_MESH,
    compiler_params=_SC_PARAMS,
    scratch_types=[
        pltpu.VMEM_SHARED((_PR, _DW), jnp.float32),
        pltpu.VMEM((2, _K2, _LANES), jnp.int32),
        pltpu.VMEM((_LANES, _DW), jnp.float32),
        pltpu.VMEM((_LANES, _DW), jnp.float32),
        pltpu.VMEM((640, _DW), jnp.float32),
        pltpu.VMEM((640, _EMB), jnp.float32),
        pltpu.SemaphoreType.DMA,
        pltpu.SemaphoreType.DMA,
    ],
    name="lightgcn_deg_sc",
)


def _gather_rows_body(tbl_e, tbl_s, tbl_d, idx, out, ibuf, ebuf, sbuf, dbuf,
                      gsem):
    c = lax.axis_index("c")
    s = lax.axis_index("s")
    wid = s * _NC + c
    nrows = 3
    pltpu.sync_copy(idx.at[pl.ds(wid * nrows, nrows)], ibuf)
    gathers = []
    for j in range(nrows):
        gathers.append(pltpu.async_copy(tbl_e.at[ibuf.at[j]], ebuf.at[j], gsem))
        gathers.append(pltpu.async_copy(tbl_s.at[ibuf.at[j]], sbuf.at[j], gsem))
        gathers.append(pltpu.async_copy(tbl_d.at[ibuf.at[j]], dbuf.at[j], gsem))
    for g in gathers:
        g.wait()

    def _combine(r, carry):
        for j in range(nrows):
            for h in (0, 16):
                e = ebuf[j, r, pl.ds(h, 16)]
                sv = sbuf[j, r, pl.ds(h, 16)]
                d = dbuf[j, r, pl.ds(h, 16)]
                ebuf[j, r, pl.ds(h, 16)] = (e + d * sv) * 0.25
        return carry

    lax.fori_loop(0, _LANES, _combine, 0)
    for j in range(nrows):
        pltpu.sync_copy(ebuf.at[j],
                        out.at[pl.ds(wid * nrows * _LANES + j * _LANES, _LANES)])


_gather_rows = pl.kernel(
    _gather_rows_body,
    out_type=jax.ShapeDtypeStruct((3 * _BATCH, _EMB), jnp.float32),
    mesh=_MESH,
    compiler_params=_SC_PARAMS,
    scratch_types=[
        pltpu.VMEM((3, _LANES), jnp.int32),
        pltpu.VMEM((3, _LANES, _EMB), jnp.float32),
        pltpu.VMEM((3, _LANES, _EMB), jnp.float32),
        pltpu.VMEM((3, _LANES, _EMB), jnp.float32),
        pltpu.SemaphoreType.DMA,
    ],
    name="lightgcn_batch_gather_sc",
)


_EW_ROWS = _G * _EMB // 128
_EW_BLOCK = _EW_ROWS // 8


def _ew_call(body, n_in, n_out):
    return pl.pallas_call(
        body,
        grid=(8,),
        in_specs=[pl.BlockSpec((_EW_BLOCK, 128), lambda i: (i, 0))] * n_in,
        out_specs=(pl.BlockSpec((_EW_BLOCK, 128), lambda i: (i, 0)),) * n_out,
        out_shape=(jax.ShapeDtypeStruct((_EW_ROWS, 128), jnp.float32),) * n_out,
    )


def _d_scale_body(deg_ref, e_ref, d_ref, eh_ref):
    d = lax.rsqrt(deg_ref[...] + 1e-9)
    d = jnp.where(jnp.isinf(d), 0.0, d)
    d_ref[...] = d
    eh_ref[...] = d * e_ref[...]


def _boundary_body(e_ref, s_ref, d_ref, en_ref, eh_ref):
    en = e_ref[...] + d_ref[...] * s_ref[...]
    en_ref[...] = en
    eh_ref[...] = d_ref[...] * en


def kernel(E0, users, pos_items, neg_items, user_idx, item_idx):
    f32 = jnp.float32
    i32 = jnp.int32
    ui = user_idx.astype(i32)
    ii = item_idx.astype(i32)

    pad_idx = jnp.full((_EP - _N_INTER,), _DUMMY, i32)
    uidx = jnp.concatenate([ui, pad_idx]).reshape(_ROWS, _LANES)
    iidx = jnp.concatenate([ii, pad_idx]).reshape(_ROWS, _LANES)
    pad2_idx = jnp.full((_EP2 - _N_INTER,), _DUMMY, i32)
    uidx2 = jnp.concatenate([ui, pad2_idx]).reshape(_ROWS2, _LANES)
    iidx2 = jnp.concatenate([ii, pad2_idx]).reshape(_ROWS2, _LANES)

    zpad = jnp.zeros((_PR - _NU, _EMB), f32)
    E_2 = jnp.stack([
        jnp.concatenate([E0[:_NU].astype(f32), zpad]),
        jnp.concatenate([E0[_NU:].astype(f32), zpad]),
    ])

    ew = lambda x: x.reshape(_EW_ROWS, 128)
    un2 = lambda x: x.reshape(_NC, _PR, _EMB)

    deg_g = ew(_deg(uidx2, iidx2))
    e_cur = ew(E_2)
    d_g, ehat = _ew_call(_d_scale_body, 2, 2)(deg_g, e_cur)
    for layer in range(_N_LAYERS):
        S = ew(_spmm(un2(ehat), uidx, iidx))
        if layer < _N_LAYERS - 1:
            e_cur, ehat = _ew_call(_boundary_body, 3, 2)(e_cur, S, d_g)

    idx_all = jnp.concatenate([
        users.astype(i32),
        pos_items.astype(i32) + _PR,
        neg_items.astype(i32) + _PR,
    ]).reshape(3 * _BATCH // _LANES, _LANES)
    res = _gather_rows(e_cur.reshape(_G, _EMB), S.reshape(_G, _EMB),
                       d_g.reshape(_G, _EMB), idx_all)
    return res[:_BATCH], res[_BATCH:2 * _BATCH], res[2 * _BATCH:]
